# baseline (speedup 1.0000x reference)
import functools

import jax
import jax.numpy as jnp
from jax.experimental import pallas as pl
from jax.experimental.pallas import tpu as pltpu

_NUM_SEGMENTS = 256
_WIN = 16
_CHUNK = 2500
_SUPER = 25000


def _final(out_ref, acc_ref, asum_ref, wm_ref, bm_ref, B):
    denom = asum_ref[...]
    denom = jnp.where(denom == 0.0, 1.0, denom)
    pooled = jax.lax.dot_general(
        acc_ref[...], wm_ref[...], (((1,), (0,)), ((), ())),
        preferred_element_type=jnp.float32)
    out_ref[...] = pooled / denom + bm_ref[...]


def _fast_body(G, Nb, C, B, S, bases_ref, x_ref, pos_ref, bid_ref,
               wg_ref, bg_ref, wm_ref, bm_ref, out_ref, acc_ref, asum_ref):
    i = pl.program_id(0)
    Kc = Nb // C

    @pl.when(i == 0)
    def _():
        acc_ref[...] = jnp.zeros_like(acc_ref)
        asum_ref[...] = jnp.zeros_like(asum_ref)

    for k in range(Kc):
        xb = x_ref[pl.ds(k * C, C), :]
        a_t = jax.lax.dot_general(wg_ref[...], xb, (((0,), (1,)), ((), ())),
                                  preferred_element_type=jnp.float32)
        alpha_t = (pos_ref[0, :, pl.ds(k * C, C)]
                   * jnp.exp(a_t + bg_ref[...]))
        bid = bid_ref[0, :, pl.ds(k * C, C)]
        base = bases_ref[i * Kc + k]
        sel = (jax.lax.broadcasted_iota(jnp.int32, (S, C), 0) + base) == bid
        ohw = jnp.where(sel, alpha_t, 0.0)
        upd = jax.lax.dot_general(ohw, xb, (((1,), (0,)), ((), ())),
                                  preferred_element_type=jnp.float32)
        acc_ref[pl.ds(base, S), :] += upd
        asum_ref[pl.ds(base, S), :] += jnp.sum(ohw, axis=1, keepdims=True)

    @pl.when(i == G - 1)
    def _():
        _final(out_ref, acc_ref, asum_ref, wm_ref, bm_ref, B)


def _body(G, Nb, B, N, S, bases_ref, x_ref, pos_ref, bid_ref,
          wg_ref, bg_ref, wm_ref, bm_ref, out_ref, acc_ref, asum_ref):
    i = pl.program_id(0)

    @pl.when(i == 0)
    def _():
        acc_ref[...] = jnp.zeros_like(acc_ref)
        asum_ref[...] = jnp.zeros_like(asum_ref)

    xb = x_ref[...]
    if N % Nb != 0:
        valid_col = (jax.lax.broadcasted_iota(jnp.int32, (Nb, 1), 0)
                     + i * Nb) < N
        xb = jnp.where(valid_col, xb, 0.0)
    a_t = jax.lax.dot_general(wg_ref[...], xb, (((0,), (1,)), ((), ())),
                              preferred_element_type=jnp.float32)
    alpha_t = pos_ref[0] * jnp.exp(a_t + bg_ref[...])
    if N % Nb != 0:
        valid = (jax.lax.broadcasted_iota(jnp.int32, (1, Nb), 1)
                 + i * Nb) < N
        alpha_t = jnp.where(valid, alpha_t, 0.0)
    bid = bid_ref[0]
    base = bases_ref[i] if S < B else 0
    sel = (jax.lax.broadcasted_iota(jnp.int32, (S, Nb), 0) + base) == bid
    ohw = jnp.where(sel, alpha_t, 0.0)
    upd = jax.lax.dot_general(ohw, xb, (((1,), (0,)), ((), ())),
                              preferred_element_type=jnp.float32)
    acc_ref[pl.ds(base, S), :] += upd
    asum_ref[pl.ds(base, S), :] += jnp.sum(ohw, axis=1, keepdims=True)

    @pl.when(i == G - 1)
    def _():
        _final(out_ref, acc_ref, asum_ref, wm_ref, bm_ref, B)


def _specs(Nb, B, D):
    return dict(
        in_specs=[
            pl.BlockSpec(memory_space=pltpu.SMEM),
            pl.BlockSpec((Nb, D), lambda i: (i, 0)),
            pl.BlockSpec((1, 1, Nb), lambda i: (i, 0, 0)),
            pl.BlockSpec((1, 1, Nb), lambda i: (i, 0, 0)),
            pl.BlockSpec((D, 1), lambda i: (0, 0)),
            pl.BlockSpec((1, 1), lambda i: (0, 0)),
            pl.BlockSpec((D, D), lambda i: (0, 0)),
            pl.BlockSpec((1, D), lambda i: (0, 0)),
        ],
        out_specs=pl.BlockSpec((B, D), lambda i: (0, 0)),
        out_shape=jax.ShapeDtypeStruct((B, D), jnp.float32),
        scratch_shapes=[pltpu.VMEM((B, D), jnp.float32),
                        pltpu.VMEM((B, 1), jnp.float32)],
    )


def _make_fast(G, Nb, C, B, S, D):
    return pl.pallas_call(functools.partial(_fast_body, G, Nb, C, B, S),
                          grid=(G,), **_specs(Nb, B, D))


def _make_single(G, Nb, B, N, S, D):
    return pl.pallas_call(functools.partial(_body, G, Nb, B, N, S),
                          grid=(G,), **_specs(Nb, B, D))


def _ranges(batch_index, G, Nb, B, S):
    bid_r = batch_index.reshape(G, Nb)
    firsts = bid_r[:, 0]
    lasts = bid_r[:, -1]
    bases = jnp.minimum((firsts // 8) * 8, B - S)
    all_small = jnp.all(lasts - bases < S)
    return bases, all_small


def kernel(x, edge_index, pos, batch_index, W_gate, b_gate, W_msg, b_msg):
    del edge_index
    N, D = x.shape
    B = _NUM_SEGMENTS
    bg2 = b_gate.reshape(1, 1)
    bm2 = b_msg.reshape(1, D)

    if N % _SUPER == 0:
        G, Nb, C, S = N // _SUPER, _SUPER, _CHUNK, _WIN
        bases, all_small = _ranges(batch_index, N // C, C, B, S)
        pos3 = pos.reshape(G, 1, Nb)
        bid3 = batch_index.reshape(G, 1, Nb)
        ops = (bases, x, pos3, bid3, W_gate, bg2, W_msg, bm2)
        Gs, Nbs = N // 5000, 5000
        return jax.lax.cond(
            all_small,
            lambda o: _make_fast(G, Nb, C, B, S, D)(*o),
            lambda o: _make_single(Gs, Nbs, B, N, B, D)(
                o[0][:Gs], o[1], o[2].reshape(Gs, 1, Nbs),
                o[3].reshape(Gs, 1, Nbs), *o[4:]),
            ops)

    S = 32
    Nb = 2000 if N % 2000 == 0 else 1024
    G = -(-N // Nb)
    pad = G * Nb - N
    if pad:
        pos = jnp.pad(pos, (0, pad))
        batch_index = jnp.pad(batch_index, (0, pad), mode="edge")
    bases, all_small = _ranges(batch_index, G, Nb, B, S)
    pos3 = pos.reshape(G, 1, Nb)
    bid3 = batch_index.reshape(G, 1, Nb)
    ops = (bases, x, pos3, bid3, W_gate, bg2, W_msg, bm2)
    return jax.lax.cond(
        all_small,
        lambda o: _make_single(G, Nb, B, N, S, D)(*o),
        lambda o: _make_single(G, Nb, B, N, B, D)(*o),
        ops)


# speedup vs baseline: 1.1372x; 1.1372x over previous
import functools

import jax
import jax.numpy as jnp
from jax.experimental import pallas as pl
from jax.experimental.pallas import tpu as pltpu

_NUM_SEGMENTS = 256
_WIN = 16
_CHUNK = 2500
_SUPER = 25000


def _final(out_ref, acc_ref, asum_ref, wm_ref, bm_ref, B):
    denom = asum_ref[...]
    denom = jnp.where(denom == 0.0, 1.0, denom)
    pooled = jax.lax.dot_general(
        acc_ref[...], wm_ref[...], (((1,), (0,)), ((), ())),
        preferred_element_type=jnp.float32)
    out_ref[...] = pooled / denom + bm_ref[...]


def _fast_body(G, Nb, C, B, S, bases_ref, x_ref, pos_ref, bid_ref,
               wg_ref, bg_ref, wm_ref, bm_ref, out_ref, acc_ref, asum_ref):
    i = pl.program_id(0)
    Kc = Nb // C

    @pl.when(i == 0)
    def _():
        acc_ref[...] = jnp.zeros_like(acc_ref)
        asum_ref[...] = jnp.zeros_like(asum_ref)

    a_t = jax.lax.dot_general(wg_ref[...], x_ref[...],
                              (((0,), (1,)), ((), ())),
                              preferred_element_type=jnp.float32)
    alpha_all = pos_ref[0] * jnp.exp(a_t + bg_ref[...])
    ones_c = jnp.ones((C, 1), dtype=jnp.float32)
    for k in range(Kc):
        xb = x_ref[pl.ds(k * C, C), :]
        alpha_t = alpha_all[:, k * C:(k + 1) * C]
        bid = bid_ref[0, :, pl.ds(k * C, C)]
        base = bases_ref[i * Kc + k]
        sel = (jax.lax.broadcasted_iota(jnp.int32, (S, C), 0) + base) == bid
        ohw = jnp.where(sel, alpha_t, 0.0)
        upd = jax.lax.dot_general(ohw, xb, (((1,), (0,)), ((), ())),
                                  preferred_element_type=jnp.float32)
        acc_ref[pl.ds(base, S), :] += upd
        asum_ref[pl.ds(base, S), :] += jax.lax.dot_general(
            ohw, ones_c, (((1,), (0,)), ((), ())),
            preferred_element_type=jnp.float32)

    @pl.when(i == G - 1)
    def _():
        _final(out_ref, acc_ref, asum_ref, wm_ref, bm_ref, B)


def _body(G, Nb, B, N, S, bases_ref, x_ref, pos_ref, bid_ref,
          wg_ref, bg_ref, wm_ref, bm_ref, out_ref, acc_ref, asum_ref):
    i = pl.program_id(0)

    @pl.when(i == 0)
    def _():
        acc_ref[...] = jnp.zeros_like(acc_ref)
        asum_ref[...] = jnp.zeros_like(asum_ref)

    xb = x_ref[...]
    if N % Nb != 0:
        valid_col = (jax.lax.broadcasted_iota(jnp.int32, (Nb, 1), 0)
                     + i * Nb) < N
        xb = jnp.where(valid_col, xb, 0.0)
    a_t = jax.lax.dot_general(wg_ref[...], xb, (((0,), (1,)), ((), ())),
                              preferred_element_type=jnp.float32)
    alpha_t = pos_ref[0] * jnp.exp(a_t + bg_ref[...])
    if N % Nb != 0:
        valid = (jax.lax.broadcasted_iota(jnp.int32, (1, Nb), 1)
                 + i * Nb) < N
        alpha_t = jnp.where(valid, alpha_t, 0.0)
    bid = bid_ref[0]
    base = bases_ref[i] if S < B else 0
    sel = (jax.lax.broadcasted_iota(jnp.int32, (S, Nb), 0) + base) == bid
    ohw = jnp.where(sel, alpha_t, 0.0)
    upd = jax.lax.dot_general(ohw, xb, (((1,), (0,)), ((), ())),
                              preferred_element_type=jnp.float32)
    acc_ref[pl.ds(base, S), :] += upd
    asum_ref[pl.ds(base, S), :] += jnp.sum(ohw, axis=1, keepdims=True)

    @pl.when(i == G - 1)
    def _():
        _final(out_ref, acc_ref, asum_ref, wm_ref, bm_ref, B)


def _specs(Nb, B, D):
    return dict(
        in_specs=[
            pl.BlockSpec(memory_space=pltpu.SMEM),
            pl.BlockSpec((Nb, D), lambda i: (i, 0)),
            pl.BlockSpec((1, 1, Nb), lambda i: (i, 0, 0)),
            pl.BlockSpec((1, 1, Nb), lambda i: (i, 0, 0)),
            pl.BlockSpec((D, 1), lambda i: (0, 0)),
            pl.BlockSpec((1, 1), lambda i: (0, 0)),
            pl.BlockSpec((D, D), lambda i: (0, 0)),
            pl.BlockSpec((1, D), lambda i: (0, 0)),
        ],
        out_specs=pl.BlockSpec((B, D), lambda i: (0, 0)),
        out_shape=jax.ShapeDtypeStruct((B, D), jnp.float32),
        scratch_shapes=[pltpu.VMEM((B, D), jnp.float32),
                        pltpu.VMEM((B, 1), jnp.float32)],
    )


def _make_fast(G, Nb, C, B, S, D):
    return pl.pallas_call(functools.partial(_fast_body, G, Nb, C, B, S),
                          grid=(G,), **_specs(Nb, B, D))


def _make_single(G, Nb, B, N, S, D):
    return pl.pallas_call(functools.partial(_body, G, Nb, B, N, S),
                          grid=(G,), **_specs(Nb, B, D))


def _ranges(batch_index, G, Nb, B, S):
    bid_r = batch_index.reshape(G, Nb)
    firsts = bid_r[:, 0]
    lasts = bid_r[:, -1]
    bases = jnp.minimum((firsts // 8) * 8, B - S)
    all_small = jnp.all(lasts - bases < S)
    return bases, all_small


def kernel(x, edge_index, pos, batch_index, W_gate, b_gate, W_msg, b_msg):
    del edge_index
    N, D = x.shape
    B = _NUM_SEGMENTS
    bg2 = b_gate.reshape(1, 1)
    bm2 = b_msg.reshape(1, D)

    if N % _SUPER == 0:
        G, Nb, C, S = N // _SUPER, _SUPER, _CHUNK, _WIN
        bases, all_small = _ranges(batch_index, N // C, C, B, S)
        pos3 = pos.reshape(G, 1, Nb)
        bid3 = batch_index.reshape(G, 1, Nb)
        ops = (bases, x, pos3, bid3, W_gate, bg2, W_msg, bm2)
        Gs, Nbs = N // 5000, 5000
        return jax.lax.cond(
            all_small,
            lambda o: _make_fast(G, Nb, C, B, S, D)(*o),
            lambda o: _make_single(Gs, Nbs, B, N, B, D)(
                o[0][:Gs], o[1], o[2].reshape(Gs, 1, Nbs),
                o[3].reshape(Gs, 1, Nbs), *o[4:]),
            ops)

    S = 32
    Nb = 2000 if N % 2000 == 0 else 1024
    G = -(-N // Nb)
    pad = G * Nb - N
    if pad:
        pos = jnp.pad(pos, (0, pad))
        batch_index = jnp.pad(batch_index, (0, pad), mode="edge")
    bases, all_small = _ranges(batch_index, G, Nb, B, S)
    pos3 = pos.reshape(G, 1, Nb)
    bid3 = batch_index.reshape(G, 1, Nb)
    ops = (bases, x, pos3, bid3, W_gate, bg2, W_msg, bm2)
    return jax.lax.cond(
        all_small,
        lambda o: _make_single(G, Nb, B, N, S, D)(*o),
        lambda o: _make_single(G, Nb, B, N, B, D)(*o),
        ops)
